# baseline (speedup 1.0000x reference)
import functools

import jax
import jax.numpy as jnp
from jax import lax
from jax.experimental import pallas as pl
from jax.experimental.pallas import tpu as pltpu
from jax.experimental.pallas import tpu_sc as plsc


def _mm_body(xa_ref, xb_ref, w_ref, o_ref, *, r, d_out):
    ha = jnp.dot(xa_ref[...], w_ref[...], preferred_element_type=jnp.float32)
    hb = jnp.dot(xb_ref[...], w_ref[...], preferred_element_type=jnp.float32)
    for ri in range(r):
        sl = slice(ri * d_out, (ri + 1) * d_out)
        o_ref[ri] = jnp.concatenate([ha[:, sl], hb[:, sl]], axis=-1)


def _rel_matmul(x, w, r, block_n):
    n, d_in = x.shape
    r_d_out = w.shape[1]
    d_out = r_d_out // r
    nb = n // 2 // block_n
    return pl.pallas_call(
        functools.partial(_mm_body, r=r, d_out=d_out),
        grid=(nb,),
        in_specs=[
            pl.BlockSpec((block_n, d_in), lambda i: (i, 0)),
            pl.BlockSpec((block_n, d_in), lambda i, nb=nb: (i + nb, 0)),
            pl.BlockSpec((d_in, r_d_out), lambda i: (0, 0)),
        ],
        out_specs=pl.BlockSpec((r, block_n, 128), lambda i: (0, i, 0)),
        out_shape=jax.ShapeDtypeStruct((r, n // 2, 128), jnp.float32),
    )(x, x, w)


def _relu_mm_body(p_ref, w_ref, o_ref, *, d_h):
    half = p_ref.shape[0] // 2
    r = w_ref.shape[0]
    q = half // 4
    h = jnp.maximum(p_ref[:half] + p_ref[half:], 0.0)
    for ri in range(r):
        oa = jnp.dot(h[:, :d_h], w_ref[ri],
                     preferred_element_type=jnp.float32)
        ob = jnp.dot(h[:, d_h:], w_ref[ri],
                     preferred_element_type=jnp.float32)
        o = jnp.concatenate([oa, ob], axis=-1)
        o_ref[ri] = jnp.concatenate([o[0:q], o[q:2 * q], o[2 * q:3 * q],
                                     o[3 * q:4 * q]], axis=-1)


def _relu_sum_matmul(p, w):
    r, d_h, d_out = w.shape
    rows = p.shape[0]
    half = rows // 2
    return pl.pallas_call(
        functools.partial(_relu_mm_body, d_h=d_h),
        grid=(1,),
        in_specs=[
            pl.BlockSpec((rows, 128), lambda i: (0, 0)),
            pl.BlockSpec((r, d_h, d_out), lambda i: (0, 0, 0)),
        ],
        out_specs=pl.BlockSpec((r, half // 4, 128), lambda i: (0, 0, 0)),
        out_shape=jax.ShapeDtypeStruct((r, half // 4, 128), jnp.float32),
    )(p, w)


def _pair_sum_body(p_ref, o_ref):
    half = p_ref.shape[0] // 2
    o_ref[...] = p_ref[:half] + p_ref[half:]


def _pair_sum(p):
    rows = p.shape[0]
    return pl.pallas_call(
        _pair_sum_body,
        grid=(1,),
        in_specs=[pl.BlockSpec((rows, 128), lambda i: (0, 0))],
        out_specs=pl.BlockSpec((rows // 2, 128), lambda i: (0, 0)),
        out_shape=jax.ShapeDtypeStruct((rows // 2, 128), jnp.float32),
    )(p)


_CHUNK = 64


def _sc_gather_scatter(table, ei4, et3, n_nodes, d, gidx_fn, nbuf):
    info = plsc.get_sparse_core_info()
    nc, ns = info.num_cores, info.num_subcores
    nw = nc * ns
    nchunk = ei4.shape[2]
    epw = nchunk * _CHUNK
    assert nchunk % nbuf == 0
    ngroups = nchunk // nbuf
    rpt = (n_nodes // ns) // 8 * 8
    rem = n_nodes - rpt * ns
    assert rem % 8 == 0 and rem <= rpt
    zr = 208
    assert rpt % zr == 0
    nz = rpt // zr

    @functools.partial(
        pl.kernel,
        out_type=jax.ShapeDtypeStruct((nc, n_nodes, d), jnp.float32),
        mesh=plsc.VectorSubcoreMesh(core_axis_name="c", subcore_axis_name="s"),
        scratch_types=[
            pltpu.VMEM((nchunk, _CHUNK), jnp.int32),
            pltpu.VMEM((nchunk, _CHUNK), jnp.int32),
            pltpu.VMEM((nchunk, _CHUNK), jnp.int32),
            [pltpu.VMEM((_CHUNK, d), jnp.float32) for _ in range(nbuf)],
            pltpu.VMEM((zr, d), jnp.float32),
            pltpu.VMEM_SHARED((n_nodes + 8, d), jnp.float32),
            [pltpu.SemaphoreType.DMA for _ in range(nbuf)],
        ],
        compiler_params=pltpu.CompilerParams(use_tc_tiling_on_sc=False),
    )
    def k(table_h, ei4_h, et3_h, out_h,
          srcv, gidx, dstv, rows, stage, acc, gsem):
        c = lax.axis_index("c")
        s = lax.axis_index("s")
        wid = s * nc + c

        pltpu.sync_copy(ei4_h.at[0].at[wid], srcv)
        pltpu.sync_copy(et3_h.at[wid], gidx)
        pltpu.sync_copy(ei4_h.at[1].at[wid], dstv)

        def irow(j):
            for kk in range(_CHUNK // 16):
                csl = pl.ds(kk * 16, 16)
                gidx[j, csl] = gidx_fn(gidx[j, csl], srcv[j, csl])

        def zrow(i, carry):
            for jj in range(d // 16):
                stage[i, pl.ds(jj * 16, 16)] = jnp.zeros((16,), jnp.float32)
            return carry

        lax.fori_loop(0, zr, zrow, 0)
        r0 = pl.multiple_of(s * rpt, 8)
        for t in range(nz):
            pltpu.sync_copy(stage, acc.at[pl.ds(r0 + t * zr, zr)])

        @pl.when(s == 0)
        def _():
            pltpu.sync_copy(stage.at[pl.ds(0, rem)],
                            acc.at[pl.ds(ns * rpt, rem)])

        plsc.subcore_barrier()

        for b in range(nbuf):
            irow(b)
            pltpu.async_copy(table_h.at[gidx.at[b]], rows[b], gsem[b])

        def outer(g, carry):
            for b in range(nbuf):
                j = g * nbuf + b

                @pl.when(g < ngroups - 1)
                def _():
                    irow(j + nbuf)

                pltpu.make_async_copy(
                    table_h.at[gidx.at[j]], rows[b], gsem[b]).wait()
                pltpu.sync_copy(rows[b], acc.at[dstv.at[j]], add=True)

                @pl.when(g < ngroups - 1)
                def _():
                    pltpu.async_copy(
                        table_h.at[gidx.at[j + nbuf]], rows[b], gsem[b])

            return carry

        lax.fori_loop(0, ngroups, outer, 0)
        plsc.subcore_barrier()

        for t in range(nz):
            pltpu.sync_copy(acc.at[pl.ds(r0 + t * zr, zr)], stage)
            pltpu.sync_copy(stage, out_h.at[c].at[pl.ds(r0 + t * zr, zr)])

        @pl.when(s == 0)
        def _():
            pltpu.sync_copy(acc.at[pl.ds(ns * rpt, rem)],
                            stage.at[pl.ds(0, rem)])
            pltpu.sync_copy(stage.at[pl.ds(0, rem)],
                            out_h.at[c].at[pl.ds(ns * rpt, rem)])

    return k(table, ei4, et3)


def kernel(x, edge_index, edge_type, W1, W2):
    n, _ = x.shape
    r, _, d_h = W1.shape
    d_out = W2.shape[2]

    info = plsc.get_sparse_core_info()
    nw = info.num_cores * info.num_subcores
    e = edge_index.shape[1]
    nchunk = -(-e // (nw * _CHUNK))
    nchunk = -(-nchunk // 20) * 20
    e_pad = nw * nchunk * _CHUNK
    pad = e_pad - e
    ei_fill = jnp.stack([jnp.zeros((pad,), jnp.int32),
                         jnp.full((pad,), n, jnp.int32)])
    shape3 = (nw, nchunk, _CHUNK)
    ei4 = jnp.concatenate([edge_index, ei_fill], axis=1).reshape(
        (2,) + shape3)
    et3 = jnp.concatenate([edge_type, jnp.zeros((pad,), jnp.int32)]).reshape(
        shape3)

    half = n // 2
    quar = n // 8

    one = jnp.int32(1)
    zero = jnp.int32(0)

    def gidx1(et, src):
        hi = jnp.where(src >= half, one, zero)
        return et * n + 2 * (src - half * hi) + hi

    def gidx2(et, src):
        t = src >> 1
        par = src & 1
        k = (jnp.where(t >= quar, one, zero)
             + jnp.where(t >= 2 * quar, one, zero)
             + jnp.where(t >= 3 * quar, one, zero))
        m = t - k * quar
        return (et * quar + m) * 8 + 2 * k + par

    w1cat = jnp.transpose(W1, (1, 0, 2)).reshape(W1.shape[1], r * d_h)
    t1 = _rel_matmul(x, w1cat, r, block_n=1000)
    p1 = _sc_gather_scatter(t1.reshape(r * n, d_h), ei4, et3, n, d_h,
                            gidx1, nbuf=10)
    t2 = _relu_sum_matmul(p1.reshape(-1, 128), W2)
    p2 = _sc_gather_scatter(t2.reshape(r * n, d_out), ei4, et3,
                            n, d_out, gidx2, nbuf=20)
    return _pair_sum(p2.reshape(-1, 128)).reshape(n, d_out)


# speedup vs baseline: 2.4764x; 2.4764x over previous
import functools

import jax
import jax.numpy as jnp
from jax import lax
from jax.experimental import pallas as pl
from jax.experimental.pallas import tpu as pltpu
from jax.experimental.pallas import tpu_sc as plsc


def _mm_body(xa_ref, xb_ref, w_ref, o_ref, *, r, d_out):
    ha = jnp.dot(xa_ref[...], w_ref[...], preferred_element_type=jnp.float32)
    hb = jnp.dot(xb_ref[...], w_ref[...], preferred_element_type=jnp.float32)
    for ri in range(r):
        sl = slice(ri * d_out, (ri + 1) * d_out)
        o_ref[ri] = jnp.concatenate([ha[:, sl], hb[:, sl]], axis=-1)


def _rel_matmul(x, w, r, block_n):
    n, d_in = x.shape
    r_d_out = w.shape[1]
    d_out = r_d_out // r
    nb = n // 2 // block_n
    return pl.pallas_call(
        functools.partial(_mm_body, r=r, d_out=d_out),
        grid=(nb,),
        in_specs=[
            pl.BlockSpec((block_n, d_in), lambda i: (i, 0)),
            pl.BlockSpec((block_n, d_in), lambda i, nb=nb: (i + nb, 0)),
            pl.BlockSpec((d_in, r_d_out), lambda i: (0, 0)),
        ],
        out_specs=pl.BlockSpec((r, block_n, 128), lambda i: (0, i, 0)),
        out_shape=jax.ShapeDtypeStruct((r, n // 2, 128), jnp.float32),
    )(x, x, w)


def _relu_mm_body(p_ref, w_ref, o_ref, *, d_h):
    half = p_ref.shape[0] // 2
    r = w_ref.shape[0]
    q = half // 4
    h = jnp.maximum(p_ref[:half] + p_ref[half:], 0.0)
    for ri in range(r):
        oa = jnp.dot(h[:, :d_h], w_ref[ri],
                     preferred_element_type=jnp.float32)
        ob = jnp.dot(h[:, d_h:], w_ref[ri],
                     preferred_element_type=jnp.float32)
        o = jnp.concatenate([oa, ob], axis=-1)
        o_ref[ri] = jnp.concatenate([o[0:q], o[q:2 * q], o[2 * q:3 * q],
                                     o[3 * q:4 * q]], axis=-1)


def _relu_sum_matmul(p, w):
    r, d_h, d_out = w.shape
    rows = p.shape[0]
    half = rows // 2
    return pl.pallas_call(
        functools.partial(_relu_mm_body, d_h=d_h),
        grid=(1,),
        in_specs=[
            pl.BlockSpec((rows, 128), lambda i: (0, 0)),
            pl.BlockSpec((r, d_h, d_out), lambda i: (0, 0, 0)),
        ],
        out_specs=pl.BlockSpec((r, half // 4, 128), lambda i: (0, 0, 0)),
        out_shape=jax.ShapeDtypeStruct((r, half // 4, 128), jnp.float32),
    )(p, w)


def _pair_sum_body(p_ref, o_ref):
    half = p_ref.shape[0] // 2
    o_ref[...] = p_ref[:half] + p_ref[half:]


def _pair_sum(p):
    rows = p.shape[0]
    return pl.pallas_call(
        _pair_sum_body,
        grid=(1,),
        in_specs=[pl.BlockSpec((rows, 128), lambda i: (0, 0))],
        out_specs=pl.BlockSpec((rows // 2, 128), lambda i: (0, 0)),
        out_shape=jax.ShapeDtypeStruct((rows // 2, 128), jnp.float32),
    )(p)


_CHUNK = 64


def _sc_gather_scatter(table, ei4, et3, n_nodes, d, gidx_fn, nbuf):
    info = plsc.get_sparse_core_info()
    nc, ns = info.num_cores, info.num_subcores
    nw = nc * ns
    nchunk = ei4.shape[2]
    epw = nchunk * _CHUNK
    assert nchunk % nbuf == 0
    ngroups = nchunk // nbuf
    rpt = (n_nodes // ns) // 8 * 8
    rem = n_nodes - rpt * ns
    assert rem % 8 == 0 and rem <= rpt
    zr = 208
    assert rpt % zr == 0
    nz = rpt // zr

    @functools.partial(
        pl.kernel,
        out_type=jax.ShapeDtypeStruct((nc, n_nodes, d), jnp.float32),
        mesh=plsc.VectorSubcoreMesh(core_axis_name="c", subcore_axis_name="s"),
        scratch_types=[
            pltpu.VMEM((nchunk, _CHUNK), jnp.int32),
            pltpu.VMEM((nchunk, _CHUNK), jnp.int32),
            pltpu.VMEM((nchunk, _CHUNK), jnp.int32),
            [pltpu.VMEM((_CHUNK, d), jnp.float32) for _ in range(nbuf)],
            pltpu.VMEM((zr, d), jnp.float32),
            pltpu.VMEM_SHARED((n_nodes + 8, d), jnp.float32),
            [pltpu.SemaphoreType.DMA for _ in range(nbuf)],
        ],
        compiler_params=pltpu.CompilerParams(use_tc_tiling_on_sc=False),
    )
    def k(table_h, ei4_h, et3_h, out_h,
          srcv, gidx, dstv, rows, stage, acc, gsem):
        c = lax.axis_index("c")
        s = lax.axis_index("s")
        wid = s * nc + c

        pltpu.sync_copy(ei4_h.at[0].at[wid], srcv)
        pltpu.sync_copy(et3_h.at[wid], gidx)
        pltpu.sync_copy(ei4_h.at[1].at[wid], dstv)

        def irow(j):
            for kk in range(_CHUNK // 16):
                csl = pl.ds(kk * 16, 16)
                gidx[j, csl] = gidx_fn(gidx[j, csl], srcv[j, csl])

        def zrow(i, carry):
            for jj in range(d // 16):
                stage[i, pl.ds(jj * 16, 16)] = jnp.zeros((16,), jnp.float32)
            return carry

        lax.fori_loop(0, zr, zrow, 0)
        r0 = pl.multiple_of(s * rpt, 8)
        for t in range(nz):
            pltpu.sync_copy(stage, acc.at[pl.ds(r0 + t * zr, zr)])

        @pl.when(s == 0)
        def _():
            pltpu.sync_copy(stage.at[pl.ds(0, rem)],
                            acc.at[pl.ds(ns * rpt, rem)])

        plsc.subcore_barrier()

        for b in range(nbuf):
            irow(b)
            pltpu.async_copy(table_h.at[gidx.at[b]], rows[b], gsem[b])

        def outer(g, carry):
            for b in range(nbuf):
                j = g * nbuf + b

                @pl.when(g < ngroups - 1)
                def _():
                    irow(j + nbuf)

                pltpu.make_async_copy(
                    table_h.at[gidx.at[j]], rows[b], gsem[b]).wait()
                pltpu.sync_copy(rows[b], acc.at[dstv.at[j]], add=True)

                @pl.when(g < ngroups - 1)
                def _():
                    pltpu.async_copy(
                        table_h.at[gidx.at[j + nbuf]], rows[b], gsem[b])

            return carry

        lax.fori_loop(0, ngroups, outer, 0)
        plsc.subcore_barrier()

        for t in range(nz):
            pltpu.sync_copy(acc.at[pl.ds(r0 + t * zr, zr)], stage)
            pltpu.sync_copy(stage, out_h.at[c].at[pl.ds(r0 + t * zr, zr)])

        @pl.when(s == 0)
        def _():
            pltpu.sync_copy(acc.at[pl.ds(ns * rpt, rem)],
                            stage.at[pl.ds(0, rem)])
            pltpu.sync_copy(stage.at[pl.ds(0, rem)],
                            out_h.at[c].at[pl.ds(ns * rpt, rem)])

    return k(table, ei4, et3)


def kernel(x, edge_index, edge_type, W1, W2):
    n, _ = x.shape
    r, _, d_h = W1.shape
    d_out = W2.shape[2]

    info = plsc.get_sparse_core_info()
    nw = info.num_cores * info.num_subcores
    e = edge_index.shape[1]
    nchunk = -(-e // (nw * _CHUNK))
    nchunk = -(-nchunk // 20) * 20
    e_pad = nw * nchunk * _CHUNK
    pad = e_pad - e
    fill_iota = jnp.arange(pad, dtype=jnp.int32)
    ei_fill = jnp.stack([fill_iota % n, n + (fill_iota % 8)])
    shape3 = (nw, nchunk, _CHUNK)
    ei4 = jnp.concatenate([edge_index, ei_fill], axis=1).reshape(
        (2,) + shape3)
    et3 = jnp.concatenate([edge_type, jnp.zeros((pad,), jnp.int32)]).reshape(
        shape3)

    half = n // 2
    quar = n // 8

    one = jnp.int32(1)
    zero = jnp.int32(0)

    def gidx1(et, src):
        hi = jnp.where(src >= half, one, zero)
        return et * n + 2 * (src - half * hi) + hi

    def gidx2(et, src):
        t = src >> 1
        par = src & 1
        k = (jnp.where(t >= quar, one, zero)
             + jnp.where(t >= 2 * quar, one, zero)
             + jnp.where(t >= 3 * quar, one, zero))
        m = t - k * quar
        return (et * quar + m) * 8 + 2 * k + par

    w1cat = jnp.transpose(W1, (1, 0, 2)).reshape(W1.shape[1], r * d_h)
    t1 = _rel_matmul(x, w1cat, r, block_n=1000)
    p1 = _sc_gather_scatter(t1.reshape(r * n, d_h), ei4, et3, n, d_h,
                            gidx1, nbuf=10)
    t2 = _relu_sum_matmul(p1.reshape(-1, 128), W2)
    p2 = _sc_gather_scatter(t2.reshape(r * n, d_out), ei4, et3,
                            n, d_out, gidx2, nbuf=20)
    return _pair_sum(p2.reshape(-1, 128)).reshape(n, d_out)


# speedup vs baseline: 2.6307x; 1.0623x over previous
import functools

import jax
import jax.numpy as jnp
from jax import lax
from jax.experimental import pallas as pl
from jax.experimental.pallas import tpu as pltpu
from jax.experimental.pallas import tpu_sc as plsc


def _mm_body(xa_ref, xb_ref, w_ref, o_ref, *, r, d_out):
    ha = jnp.dot(xa_ref[...], w_ref[...], preferred_element_type=jnp.float32)
    hb = jnp.dot(xb_ref[...], w_ref[...], preferred_element_type=jnp.float32)
    for ri in range(r):
        sl = slice(ri * d_out, (ri + 1) * d_out)
        o_ref[ri] = jnp.concatenate([ha[:, sl], hb[:, sl]], axis=-1)


def _rel_matmul(x, w, r, block_n):
    n, d_in = x.shape
    r_d_out = w.shape[1]
    d_out = r_d_out // r
    nb = n // 2 // block_n
    return pl.pallas_call(
        functools.partial(_mm_body, r=r, d_out=d_out),
        grid=(nb,),
        in_specs=[
            pl.BlockSpec((block_n, d_in), lambda i: (i, 0)),
            pl.BlockSpec((block_n, d_in), lambda i, nb=nb: (i + nb, 0)),
            pl.BlockSpec((d_in, r_d_out), lambda i: (0, 0)),
        ],
        out_specs=pl.BlockSpec((r, block_n, 128), lambda i: (0, i, 0)),
        out_shape=jax.ShapeDtypeStruct((r, n // 2, 128), jnp.float32),
    )(x, x, w)


def _relu_mm_body(p_ref, w_ref, o_ref, *, d_h):
    half = p_ref.shape[0] // 2
    r = w_ref.shape[0]
    q = half // 4
    h = jnp.maximum(p_ref[:half] + p_ref[half:], 0.0)
    for ri in range(r):
        oa = jnp.dot(h[:, :d_h], w_ref[ri],
                     preferred_element_type=jnp.float32)
        ob = jnp.dot(h[:, d_h:], w_ref[ri],
                     preferred_element_type=jnp.float32)
        o = jnp.concatenate([oa, ob], axis=-1)
        o_ref[ri] = jnp.concatenate([o[0:q], o[q:2 * q], o[2 * q:3 * q],
                                     o[3 * q:4 * q]], axis=-1)


def _relu_sum_matmul(p, w):
    r, d_h, d_out = w.shape
    rows = p.shape[0]
    half = rows // 2
    return pl.pallas_call(
        functools.partial(_relu_mm_body, d_h=d_h),
        grid=(1,),
        in_specs=[
            pl.BlockSpec((rows, 128), lambda i: (0, 0)),
            pl.BlockSpec((r, d_h, d_out), lambda i: (0, 0, 0)),
        ],
        out_specs=pl.BlockSpec((r, half // 4, 128), lambda i: (0, 0, 0)),
        out_shape=jax.ShapeDtypeStruct((r, half // 4, 128), jnp.float32),
    )(p, w)


def _pair_sum_body(p_ref, o_ref):
    half = p_ref.shape[0] // 2
    o_ref[...] = p_ref[:half] + p_ref[half:]


def _pair_sum(p):
    rows = p.shape[0]
    return pl.pallas_call(
        _pair_sum_body,
        grid=(1,),
        in_specs=[pl.BlockSpec((rows, 128), lambda i: (0, 0))],
        out_specs=pl.BlockSpec((rows // 2, 128), lambda i: (0, 0)),
        out_shape=jax.ShapeDtypeStruct((rows // 2, 128), jnp.float32),
    )(p)


_CHUNK = 80


def _sc_gather_scatter(table, ei4, et3, n_nodes, d, gidx_fn, nbuf):
    info = plsc.get_sparse_core_info()
    nc, ns = info.num_cores, info.num_subcores
    nw = nc * ns
    nchunk = ei4.shape[2]
    epw = nchunk * _CHUNK
    assert nchunk % nbuf == 0
    ngroups = nchunk // nbuf
    rpt = (n_nodes // ns) // 8 * 8
    rem = n_nodes - rpt * ns
    assert rem % 8 == 0 and rem <= rpt
    zr = 208
    assert rpt % zr == 0
    nz = rpt // zr

    @functools.partial(
        pl.kernel,
        out_type=jax.ShapeDtypeStruct((nc, n_nodes, d), jnp.float32),
        mesh=plsc.VectorSubcoreMesh(core_axis_name="c", subcore_axis_name="s"),
        scratch_types=[
            pltpu.VMEM((nchunk, _CHUNK), jnp.int32),
            pltpu.VMEM((nchunk, _CHUNK), jnp.int32),
            pltpu.VMEM((nchunk, _CHUNK), jnp.int32),
            [pltpu.VMEM((_CHUNK, d), jnp.float32) for _ in range(nbuf)],
            pltpu.VMEM((zr, d), jnp.float32),
            pltpu.VMEM_SHARED((n_nodes + 8, d), jnp.float32),
            [pltpu.SemaphoreType.DMA for _ in range(nbuf)],
        ],
        compiler_params=pltpu.CompilerParams(use_tc_tiling_on_sc=False),
    )
    def k(table_h, ei4_h, et3_h, out_h,
          srcv, gidx, dstv, rows, stage, acc, gsem):
        c = lax.axis_index("c")
        s = lax.axis_index("s")
        wid = s * nc + c

        pltpu.sync_copy(ei4_h.at[0].at[wid], srcv)
        pltpu.sync_copy(et3_h.at[wid], gidx)
        pltpu.sync_copy(ei4_h.at[1].at[wid], dstv)

        def irow(j):
            for kk in range(_CHUNK // 16):
                csl = pl.ds(kk * 16, 16)
                gidx[j, csl] = gidx_fn(gidx[j, csl], srcv[j, csl])

        def zrow(i, carry):
            for jj in range(d // 16):
                stage[i, pl.ds(jj * 16, 16)] = jnp.zeros((16,), jnp.float32)
            return carry

        lax.fori_loop(0, zr, zrow, 0)
        r0 = pl.multiple_of(s * rpt, 8)
        for t in range(nz):
            pltpu.sync_copy(stage, acc.at[pl.ds(r0 + t * zr, zr)])

        @pl.when(s == 0)
        def _():
            pltpu.sync_copy(stage.at[pl.ds(0, rem)],
                            acc.at[pl.ds(ns * rpt, rem)])

        plsc.subcore_barrier()

        for b in range(nbuf):
            irow(b)
            pltpu.async_copy(table_h.at[gidx.at[b]], rows[b], gsem[b])

        def outer(g, carry):
            for b in range(nbuf):
                j = g * nbuf + b

                @pl.when(g < ngroups - 1)
                def _():
                    irow(j + nbuf)

                pltpu.make_async_copy(
                    table_h.at[gidx.at[j]], rows[b], gsem[b]).wait()
                pltpu.sync_copy(rows[b], acc.at[dstv.at[j]], add=True)

                @pl.when(g < ngroups - 1)
                def _():
                    pltpu.async_copy(
                        table_h.at[gidx.at[j + nbuf]], rows[b], gsem[b])

            return carry

        lax.fori_loop(0, ngroups, outer, 0)
        plsc.subcore_barrier()

        for t in range(nz):
            pltpu.sync_copy(acc.at[pl.ds(r0 + t * zr, zr)], stage)
            pltpu.sync_copy(stage, out_h.at[c].at[pl.ds(r0 + t * zr, zr)])

        @pl.when(s == 0)
        def _():
            pltpu.sync_copy(acc.at[pl.ds(ns * rpt, rem)],
                            stage.at[pl.ds(0, rem)])
            pltpu.sync_copy(stage.at[pl.ds(0, rem)],
                            out_h.at[c].at[pl.ds(ns * rpt, rem)])

    return k(table, ei4, et3)


def kernel(x, edge_index, edge_type, W1, W2):
    n, _ = x.shape
    r, _, d_h = W1.shape
    d_out = W2.shape[2]

    info = plsc.get_sparse_core_info()
    nw = info.num_cores * info.num_subcores
    e = edge_index.shape[1]
    nchunk = -(-e // (nw * _CHUNK))
    nchunk = -(-nchunk // 5) * 5
    e_pad = nw * nchunk * _CHUNK
    pad = e_pad - e
    ei_fill = jnp.stack([jnp.zeros((pad,), jnp.int32),
                         jnp.full((pad,), n, jnp.int32)])
    shape3 = (nw, nchunk, _CHUNK)
    ei4 = jnp.concatenate([edge_index, ei_fill], axis=1).reshape(
        (2,) + shape3)
    et3 = jnp.concatenate([edge_type, jnp.zeros((pad,), jnp.int32)]).reshape(
        shape3)

    half = n // 2
    quar = n // 8

    one = jnp.int32(1)
    zero = jnp.int32(0)

    def gidx1(et, src):
        hi = jnp.where(src >= half, one, zero)
        return et * n + 2 * (src - half * hi) + hi

    def gidx2(et, src):
        t = src >> 1
        par = src & 1
        k = (jnp.where(t >= quar, one, zero)
             + jnp.where(t >= 2 * quar, one, zero)
             + jnp.where(t >= 3 * quar, one, zero))
        m = t - k * quar
        return (et * quar + m) * 8 + 2 * k + par

    w1cat = jnp.transpose(W1, (1, 0, 2)).reshape(W1.shape[1], r * d_h)
    t1 = _rel_matmul(x, w1cat, r, block_n=1000)
    p1 = _sc_gather_scatter(t1.reshape(r * n, d_h), ei4, et3, n, d_h,
                            gidx1, nbuf=5)
    t2 = _relu_sum_matmul(p1.reshape(-1, 128), W2)
    p2 = _sc_gather_scatter(t2.reshape(r * n, d_out), ei4, et3,
                            n, d_out, gidx2, nbuf=25)
    return _pair_sum(p2.reshape(-1, 128)).reshape(n, d_out)
